# baseline (speedup 1.0000x reference)
import jax
import jax.numpy as jnp
from jax.experimental import pallas as pl
from jax.experimental.pallas import tpu as pltpu


def _som_dist_kernel(x_ref, w_ref, out_ref, scratch, sems):
    R, C, D = w_ref.shape
    x = x_ref[...]
    xs = x * -2.0
    x2 = jnp.sum(x * x, axis=1, keepdims=True)
    for r in range(R):
        wr = w_ref[r]
        w2r = jnp.sum(wr * wr, axis=1)[None, :]
        xwr = jax.lax.dot_general(
            xs, wr, (((1,), (1,)), ((), ())),
            preferred_element_type=jnp.float32,
        )
        scratch[r] = (xwr + x2) + w2r
        pltpu.make_async_copy(
            scratch.at[r],
            out_ref.at[:, r, :],
            sems.at[r],
        ).start()
    for r in range(R):
        pltpu.make_async_copy(
            scratch.at[r],
            out_ref.at[:, r, :],
            sems.at[r],
        ).wait()


def kernel(x, weights):
    R, C, D = weights.shape
    B = x.shape[0]
    return pl.pallas_call(
        _som_dist_kernel,
        in_specs=[
            pl.BlockSpec(memory_space=pltpu.VMEM),
            pl.BlockSpec(memory_space=pltpu.VMEM),
        ],
        out_specs=pl.BlockSpec(memory_space=pl.ANY),
        out_shape=jax.ShapeDtypeStruct((B, R, C), jnp.float32),
        scratch_shapes=[
            pltpu.VMEM((R, B, C), jnp.float32),
            pltpu.SemaphoreType.DMA((R,)),
        ],
    )(x, weights)


# speedup vs baseline: 1.5940x; 1.5940x over previous
import jax
import jax.numpy as jnp
from jax.experimental import pallas as pl
from jax.experimental.pallas import tpu as pltpu


def _som_dist_kernel(x_ref, w_ref, out_ref):
    RB, C, D = w_ref.shape
    w = w_ref.reshape(RB * C, D)[...]
    x = x_ref[...]
    xs = x * -2.0
    xw = jax.lax.dot_general(
        xs, w, (((1,), (1,)), ((), ())),
        preferred_element_type=jnp.float32,
    )
    x2 = jnp.sum(x * x, axis=1, keepdims=True)
    ones = jnp.ones((1, D), jnp.float32)
    w2 = jax.lax.dot_general(
        ones, w * w, (((1,), (1,)), ((), ())),
        preferred_element_type=jnp.float32,
    )
    out_ref[...] = (xw + x2) + w2


def kernel(x, weights):
    R, C, D = weights.shape
    B = x.shape[0]
    N = R * C
    STEPS = 4
    RB = R // STEPS
    out = pl.pallas_call(
        _som_dist_kernel,
        grid=(STEPS,),
        in_specs=[
            pl.BlockSpec((B, D), lambda i: (0, 0)),
            pl.BlockSpec((RB, C, D), lambda i: (i, 0, 0)),
        ],
        out_specs=pl.BlockSpec((B, RB * C), lambda i: (0, i)),
        out_shape=jax.ShapeDtypeStruct((B, N), jnp.float32),
        compiler_params=pltpu.CompilerParams(
            dimension_semantics=("arbitrary",),
        ),
    )(x, weights)
    return out.reshape(B, R, C)


# speedup vs baseline: 1.8796x; 1.1792x over previous
import jax
import jax.numpy as jnp
from jax.experimental import pallas as pl


def _som_dist_kernel(x_ref, w_ref, out_ref):
    R, C, D = w_ref.shape
    w = w_ref.reshape(R * C, D)[...]
    x = x_ref[...]
    xs = x * -2.0
    xw = jax.lax.dot_general(
        xs, w, (((1,), (1,)), ((), ())),
        preferred_element_type=jnp.float32,
    )
    x2 = jnp.sum(x * x, axis=1, keepdims=True)
    ones = jnp.ones((1, D), jnp.float32)
    w2 = jax.lax.dot_general(
        ones, w * w, (((1,), (1,)), ((), ())),
        preferred_element_type=jnp.float32,
    )
    out_ref[...] = (xw + x2) + w2


def kernel(x, weights):
    R, C, D = weights.shape
    B = x.shape[0]
    N = R * C
    out = pl.pallas_call(
        _som_dist_kernel,
        out_shape=jax.ShapeDtypeStruct((B, N), jnp.float32),
    )(x, weights)
    return out.reshape(B, R, C)


# speedup vs baseline: 1.9183x; 1.0206x over previous
import jax
import jax.numpy as jnp
from jax.experimental import pallas as pl


def _som_dist_kernel(x_ref, w_ref, out_ref):
    R, C, D = w_ref.shape
    w = w_ref.reshape(R * C, D)[...]
    x = x_ref[...]
    xs = x * -2.0
    xw = jax.lax.dot_general(
        xs, w, (((1,), (1,)), ((), ())),
        preferred_element_type=jnp.float32,
    )
    x2 = jnp.sum(x * x, axis=1, keepdims=True)
    ones = jnp.ones((1, D), jnp.float32)
    w2 = jax.lax.dot_general(
        ones, w * w, (((1,), (1,)), ((), ())),
        preferred_element_type=jnp.float32,
    )
    out_ref[...] = (xw + x2) + w2


def kernel(x, weights):
    R, C, D = weights.shape
    B = x.shape[0]
    N = R * C
    from jax.experimental.pallas import tpu as pltpu
    out = pl.pallas_call(
        _som_dist_kernel,
        grid=(2,),
        in_specs=[
            pl.BlockSpec((B, D), lambda i: (0, 0)),
            pl.BlockSpec((R // 2, C, D), lambda i: (i, 0, 0)),
        ],
        out_specs=pl.BlockSpec((B, N // 2), lambda i: (0, i)),
        out_shape=jax.ShapeDtypeStruct((B, N), jnp.float32),
        compiler_params=pltpu.CompilerParams(
            dimension_semantics=("arbitrary",),
        ),
    )(x, weights)
    return out.reshape(B, R, C)
